# baseline (speedup 1.0000x reference)
import functools

import jax
import jax.numpy as jnp
from jax import lax
from jax.experimental import pallas as pl
from jax.experimental.pallas import tpu as pltpu
from jax.experimental.pallas import tpu_sc as plsc

_NC, _NS, _L = 2, 16, 16
_NW = _NC * _NS
_ROWS, _COLS = 16384, 128
_ROWS_W = _ROWS // _NW
_CROWS = 64
_NCHUNK = _ROWS_W // _CROWS
_KPR = _COLS // _L


_GATHER_DNUMS = lax.GatherDimensionNumbers(
    offset_dims=(), collapsed_slice_dims=(0,), start_index_map=(0,))


def _wrap_table16():
    i = lax.iota(jnp.int32, 16)
    return jnp.where(i == jnp.int32(10), jnp.int32(-1),
                     jnp.where(i >= jnp.int32(11), i - jnp.int32(12),
                               i - jnp.int32(1)))


def _map_vec(x, tbl):
    idx = lax.bitwise_and(x, jnp.int32(15))
    return lax.gather(tbl, idx[:, None], _GATHER_DNUMS, (1,),
                      mode=lax.GatherScatterMode.PROMISE_IN_BOUNDS)


@functools.partial(
    pl.kernel,
    mesh=plsc.VectorSubcoreMesh(core_axis_name="c", subcore_axis_name="s"),
    out_type=jax.ShapeDtypeStruct((_ROWS, _COLS), jnp.int32),
    scratch_types=[
        pltpu.VMEM((_NCHUNK, _CROWS, _COLS), jnp.int32),
        pltpu.SemaphoreType.DMA,
        pltpu.SemaphoreType.DMA,
    ],
)
def _convert(sp_hbm, out_hbm, buf, sem_in, sem_out):
    wid = lax.axis_index("s") * _NC + lax.axis_index("c")
    row0 = wid * _ROWS_W
    tbl = _wrap_table16()

    def _fire(g, _):
        pltpu.async_copy(sp_hbm.at[pl.ds(row0 + g * _CROWS, _CROWS)],
                         buf.at[g], sem_in)
        return 0
    lax.fori_loop(0, _NCHUNK, _fire, 0)

    def _chunk(g, _):
        pltpu.make_async_copy(sp_hbm.at[pl.ds(0, _CROWS)], buf.at[g],
                              sem_in).wait()

        def body(r, _):
            xs = [buf[g, r, pl.ds(k * _L, _L)] for k in range(_KPR)]
            ys = [_map_vec(x, tbl) for x in xs]
            for k, y in enumerate(ys):
                buf[g, r, pl.ds(k * _L, _L)] = y
            return 0
        lax.fori_loop(0, _CROWS, body, 0)
        pltpu.async_copy(buf.at[g],
                         out_hbm.at[pl.ds(row0 + g * _CROWS, _CROWS)],
                         sem_out)
        return 0
    lax.fori_loop(0, _NCHUNK, _chunk, 0)

    def _drain(g, _):
        pltpu.make_async_copy(buf.at[0], out_hbm.at[pl.ds(0, _CROWS)],
                              sem_out).wait()
        return 0
    lax.fori_loop(0, _NCHUNK, _drain, 0)


def kernel(species, coordinates):
    one = lax.optimization_barrier(jnp.float32(1.0))
    return (_convert(species), coordinates * one)
